# baseline (speedup 1.0000x reference)
import functools

import jax
import jax.numpy as jnp
from jax import lax
from jax.experimental import pallas as pl
from jax.experimental.pallas import tpu as pltpu
from jax.experimental.pallas import tpu_sc as plsc

N_EX = 100000
N_CAT = 1000
D = 64
S = 200
B = 4096
S2 = 2 * S
S2P = S2 + 16

_NC = 2
_NS = 16
_NW = _NC * _NS
_ROWS_PER_W = B // _NW
_NSLOT = 4
_NIDX = 4
_LANES = 16
_VECS_PER_ROW = D // _LANES

_GATHER_SPLITS = ((0, 128), (128, 72))


def _emb_body(idx_hbm, tab_hbm, ctab_hbm, ptab_hbm, out_hbm,
              idx_v, d_buf, c_tab, p_buf, *sems):
    sem_i = sems[0:_NIDX]
    sem_g = sems[_NIDX:_NIDX + _NSLOT]
    sem_o = sems[_NIDX + _NSLOT:_NIDX + 2 * _NSLOT]
    wid = lax.axis_index("s") * _NC + lax.axis_index("c")
    base = wid * _ROWS_PER_W

    pltpu.sync_copy(ctab_hbm, c_tab)
    pltpu.sync_copy(ptab_hbm, p_buf)

    def idx_cp(row, ki):
        return pltpu.make_async_copy(
            idx_hbm.at[row], idx_v.at[ki, pl.ds(0, S2)], sem_i[ki])

    def gather_cps(ki, kd):
        return [pltpu.make_async_copy(
                    tab_hbm.at[idx_v.at[ki, pl.ds(lo, ln)]],
                    d_buf.at[kd, pl.ds(lo, ln)],
                    sem_g[kd])
                for lo, ln in _GATHER_SPLITS]

    def out_cp(row, kd):
        return pltpu.make_async_copy(d_buf.at[kd], out_hbm.at[row], sem_o[kd])

    for r in range(3):
        idx_cp(base + r, r).start()
    for r in range(2):
        idx_cp(base + r, r).wait()
        for cp in gather_cps(r, r):
            cp.start()

    def group_body(g, carry):
        for k in range(_NSLOT):
            r = _NSLOT * g + k
            kd2 = (k + 2) % _NSLOT

            for cp in gather_cps(0, k):
                cp.wait()

            @pl.when(r + 2 < _ROWS_PER_W)
            def _():
                idx_cp(base + r + 2, (k + 2) % _NIDX).wait()

                @pl.when(r >= 2)
                def _():
                    out_cp(base + r - 2, kd2).wait()

                for cp in gather_cps((k + 2) % _NIDX, kd2):
                    cp.start()

            @pl.when(r + 3 < _ROWS_PER_W)
            def _():
                idx_cp(base + r + 3, (k + 3) % _NIDX).start()

            def s_block(s0, nl):
                catv = idx_v[k, pl.ds(S + s0, _LANES)]
                for l in range(nl):
                    s = s0 + l
                    coff = catv[l]
                    for j in range(_VECS_PER_ROW):
                        sl = pl.ds(j * _LANES, _LANES)
                        cvec = c_tab[pl.ds(coff + j * _LANES, _LANES)]
                        plsc.addupdate(d_buf.at[k, s, sl], cvec + p_buf[s, sl])

            @plsc.parallel_loop(0, S // _LANES, unroll=2)
            def _(g2):
                s_block(g2 * _LANES, _LANES)

            s_block((S // _LANES) * _LANES, S % _LANES)
            out_cp(base + r, k).start()
        return carry

    lax.fori_loop(0, _ROWS_PER_W // _NSLOT, group_body, 0)

    for r in range(_ROWS_PER_W - 4, _ROWS_PER_W):
        out_cp(base + r, r % _NSLOT).wait()


_emb_kernel = functools.partial(
    pl.kernel,
    out_type=jax.ShapeDtypeStruct((B, S, D), jnp.float32),
    scratch_types=(
        [pltpu.VMEM((_NIDX, S2P), jnp.int32),
         pltpu.VMEM((_NSLOT, S, D), jnp.float32),
         pltpu.VMEM((N_CAT * D,), jnp.float32),
         pltpu.VMEM((S, D), jnp.float32)]
        + [pltpu.SemaphoreType.DMA] * (_NIDX + 2 * _NSLOT)
    ),
    mesh=plsc.VectorSubcoreMesh(core_axis_name="c", subcore_axis_name="s"),
    compiler_params=pltpu.CompilerParams(use_tc_tiling_on_sc=False,
                                         needs_layout_passes=False,
                                         disable_bounds_checks=True),
)(_emb_body)


def kernel(exercises, categories, exercise_table, category_table, position_table):
    idx = jnp.concatenate(
        [exercises, categories.astype(jnp.int32) * D], axis=1)
    return _emb_kernel(idx, exercise_table, category_table.reshape(-1),
                       position_table)


# speedup vs baseline: 1.1924x; 1.1924x over previous
import functools

import jax
import jax.numpy as jnp
from jax import lax
from jax.experimental import pallas as pl
from jax.experimental.pallas import tpu as pltpu
from jax.experimental.pallas import tpu_sc as plsc

N_EX = 100000
N_CAT = 1000
D = 64
S = 200
B = 4096
S2 = 2 * S
S2P = S2 + 16

_NC = 2
_NS = 16
_NW = _NC * _NS
_ROWS_PER_W = B // _NW
_NSLOT = 4
_NIDX = 4
_LANES = 16
_VECS_PER_ROW = D // _LANES

_GATHER_SPLITS = ((0, 56), (56, 48), (104, 48), (152, 48))


def _emb_body(idx_hbm, tab_hbm, ctab_hbm, ptab_hbm, out_hbm,
              idx_v, d_buf, c_tab, p_buf, *sems):
    sem_i = sems[0:_NIDX]
    sem_g = sems[_NIDX:_NIDX + _NSLOT]
    sem_o = sems[_NIDX + _NSLOT:_NIDX + 2 * _NSLOT]
    wid = lax.axis_index("s") * _NC + lax.axis_index("c")
    base = wid * _ROWS_PER_W

    pltpu.sync_copy(ctab_hbm, c_tab)
    pltpu.sync_copy(ptab_hbm, p_buf)

    def idx_cp(row, ki):
        return pltpu.make_async_copy(
            idx_hbm.at[row], idx_v.at[ki, pl.ds(0, S2)], sem_i[ki])

    def gather_cps(ki, kd):
        return [pltpu.make_async_copy(
                    tab_hbm.at[idx_v.at[ki, pl.ds(lo, ln)]],
                    d_buf.at[kd, pl.ds(lo, ln)],
                    sem_g[kd])
                for lo, ln in _GATHER_SPLITS]

    def out_cp(row, kd):
        return pltpu.make_async_copy(d_buf.at[kd], out_hbm.at[row], sem_o[kd])

    for r in range(3):
        idx_cp(base + r, r).start()
    for r in range(2):
        idx_cp(base + r, r).wait()
        for cp in gather_cps(r, r):
            cp.start()

    def group_body(g, carry):
        for k in range(_NSLOT):
            r = _NSLOT * g + k
            kd2 = (k + 2) % _NSLOT

            for cp in gather_cps(0, k):
                cp.wait()

            @pl.when(r + 2 < _ROWS_PER_W)
            def _():
                idx_cp(base + r + 2, (k + 2) % _NIDX).wait()

                @pl.when(r >= 2)
                def _():
                    out_cp(base + r - 2, kd2).wait()

                for cp in gather_cps((k + 2) % _NIDX, kd2):
                    cp.start()

            @pl.when(r + 3 < _ROWS_PER_W)
            def _():
                idx_cp(base + r + 3, (k + 3) % _NIDX).start()

            def s_block(s0, nl):
                catv = idx_v[k, pl.ds(S + s0, _LANES)]
                for l in range(nl):
                    s = s0 + l
                    coff = catv[l]
                    for j in range(_VECS_PER_ROW):
                        sl = pl.ds(j * _LANES, _LANES)
                        cvec = c_tab[pl.ds(coff + j * _LANES, _LANES)]
                        plsc.addupdate(d_buf.at[k, s, sl], cvec + p_buf[s, sl])

            @plsc.parallel_loop(0, S // _LANES)
            def _(g2):
                s_block(g2 * _LANES, _LANES)

            s_block((S // _LANES) * _LANES, S % _LANES)
            out_cp(base + r, k).start()
        return carry

    lax.fori_loop(0, _ROWS_PER_W // _NSLOT, group_body, 0)

    for r in range(_ROWS_PER_W - 4, _ROWS_PER_W):
        out_cp(base + r, r % _NSLOT).wait()


_emb_kernel = functools.partial(
    pl.kernel,
    out_type=jax.ShapeDtypeStruct((B, S, D), jnp.float32),
    scratch_types=(
        [pltpu.VMEM((_NIDX, S2P), jnp.int32),
         pltpu.VMEM((_NSLOT, S, D), jnp.float32),
         pltpu.VMEM((N_CAT * D,), jnp.float32),
         pltpu.VMEM((S, D), jnp.float32)]
        + [pltpu.SemaphoreType.DMA] * (_NIDX + 2 * _NSLOT)
    ),
    mesh=plsc.VectorSubcoreMesh(core_axis_name="c", subcore_axis_name="s"),
    compiler_params=pltpu.CompilerParams(use_tc_tiling_on_sc=False,
                                         needs_layout_passes=False,
                                         disable_bounds_checks=True),
)(_emb_body)


def kernel(exercises, categories, exercise_table, category_table, position_table):
    idx = jnp.concatenate(
        [exercises, categories.astype(jnp.int32) * D], axis=1)
    return _emb_kernel(idx, exercise_table, category_table.reshape(-1),
                       position_table)


# speedup vs baseline: 1.2038x; 1.0096x over previous
import functools

import jax
import jax.numpy as jnp
from jax import lax
from jax.experimental import pallas as pl
from jax.experimental.pallas import tpu as pltpu
from jax.experimental.pallas import tpu_sc as plsc

N_EX = 100000
N_CAT = 1000
D = 64
S = 200
B = 4096
S2 = 2 * S
S2P = S2 + 16

_NC = 2
_NS = 16
_NW = _NC * _NS
_ROWS_PER_W = B // _NW
_NSLOT = 4
_NIDX = 4
_LANES = 16
_VECS_PER_ROW = D // _LANES

_GATHER_SPLITS = ((0, 128), (128, 72))


def _emb_body(idx_hbm, tab_hbm, ctab_hbm, ptab_hbm, out_hbm,
              idx_v, d_buf, c_tab, p_buf, *sems):
    sem_i = sems[0:_NIDX]
    sem_g = sems[_NIDX:_NIDX + _NSLOT]
    sem_o = sems[_NIDX + _NSLOT:_NIDX + 2 * _NSLOT]
    wid = lax.axis_index("s") * _NC + lax.axis_index("c")
    base = wid * _ROWS_PER_W

    pltpu.sync_copy(ctab_hbm, c_tab)
    pltpu.sync_copy(ptab_hbm, p_buf)

    def idx_cp(row, ki):
        return pltpu.make_async_copy(
            idx_hbm.at[row], idx_v.at[ki, pl.ds(0, S2)], sem_i[ki])

    def gather_cps(ki, kd):
        return [pltpu.make_async_copy(
                    tab_hbm.at[idx_v.at[ki, pl.ds(lo, ln)]],
                    d_buf.at[kd, pl.ds(lo, ln)],
                    sem_g[kd])
                for lo, ln in _GATHER_SPLITS]

    def out_cp(row, kd):
        return pltpu.make_async_copy(d_buf.at[kd], out_hbm.at[row], sem_o[kd])

    for r in range(3):
        idx_cp(base + r, r).start()
    for r in range(2):
        idx_cp(base + r, r).wait()
        for cp in gather_cps(r, r):
            cp.start()

    def group_body(g, carry):
        for k in range(_NSLOT):
            r = _NSLOT * g + k
            kd2 = (k + 2) % _NSLOT

            for cp in gather_cps(0, k):
                cp.wait()

            @pl.when(r + 2 < _ROWS_PER_W)
            def _():
                idx_cp(base + r + 2, (k + 2) % _NIDX).wait()

                @pl.when(r >= 2)
                def _():
                    out_cp(base + r - 2, kd2).wait()

                for cp in gather_cps((k + 2) % _NIDX, kd2):
                    cp.start()

            @pl.when(r + 3 < _ROWS_PER_W)
            def _():
                idx_cp(base + r + 3, (k + 3) % _NIDX).start()

            def s_block(s0, nl):
                catv = idx_v[k, pl.ds(S + s0, _LANES)]
                for l in range(nl):
                    s = s0 + l
                    coff = catv[l]
                    for j in range(_VECS_PER_ROW):
                        sl = pl.ds(j * _LANES, _LANES)
                        cvec = c_tab[pl.ds(coff + j * _LANES, _LANES)]
                        plsc.addupdate(d_buf.at[k, s, sl], cvec + p_buf[s, sl])

            @plsc.parallel_loop(0, S // _LANES)
            def _(g2):
                s_block(g2 * _LANES, _LANES)

            s_block((S // _LANES) * _LANES, S % _LANES)
            out_cp(base + r, k).start()
        return carry

    lax.fori_loop(0, _ROWS_PER_W // _NSLOT, group_body, 0)

    for r in range(_ROWS_PER_W - 4, _ROWS_PER_W):
        out_cp(base + r, r % _NSLOT).wait()


_emb_kernel = functools.partial(
    pl.kernel,
    out_type=jax.ShapeDtypeStruct((B, S, D), jnp.float32),
    scratch_types=(
        [pltpu.VMEM((_NIDX, S2P), jnp.int32),
         pltpu.VMEM((_NSLOT, S, D), jnp.float32),
         pltpu.VMEM((N_CAT * D,), jnp.float32),
         pltpu.VMEM((S, D), jnp.float32)]
        + [pltpu.SemaphoreType.DMA] * (_NIDX + 2 * _NSLOT)
    ),
    mesh=plsc.VectorSubcoreMesh(core_axis_name="c", subcore_axis_name="s"),
    compiler_params=pltpu.CompilerParams(use_tc_tiling_on_sc=False,
                                         needs_layout_passes=False,
                                         disable_bounds_checks=True),
)(_emb_body)


def kernel(exercises, categories, exercise_table, category_table, position_table):
    idx = jnp.concatenate(
        [exercises, categories.astype(jnp.int32) * D], axis=1)
    return _emb_kernel(idx, exercise_table, category_table.reshape(-1),
                       position_table)
